# baseline (speedup 1.0000x reference)
import jax
import jax.numpy as jnp
import numpy as np
from jax import lax
from jax.experimental import pallas as pl
from jax.experimental.pallas import tpu as pltpu
from jax.experimental.pallas import tpu_sc as plsc

H = W = 32
HW = H * W
NH = 16
SUB = (2 * H - 1) * (2 * W - 1)
NC, NS = 2, 16
NW = NC * NS
ROWS = NH * HW
RPW = ROWS // NW
LANES = 16
CHUNKS = HW // LANES
BATCH = 8
NBUF = 4
NBATCH = RPW // BATCH
CCH = 128
NCCH = (SUB + CCH - 1) // CCH
COLPAD = NCCH * CCH


def _body(tab_hbm, colidx_hbm, d_hbm, bsp_hbm, out_hbm,
          idx_v, col_v, d_v, bsp_v, buf_v, *sems):
    cid = lax.axis_index("c")
    sid = lax.axis_index("s")
    wid = sid * NC + cid
    head = wid // 2
    r0 = wid * RPW

    pltpu.sync_copy(colidx_hbm.at[head], idx_v)
    pltpu.sync_copy(d_hbm, d_v)
    pltpu.sync_copy(bsp_hbm.at[pl.ds(r0 * LANES, RPW * LANES)], bsp_v)
    gsem = sems[NBUF]
    cps = [pltpu.async_copy(tab_hbm.at[idx_v.at[t]],
                            col_v.at[pl.ds(t * CCH, CCH)], gsem)
           for t in range(NCCH)]
    for cp in cps:
        cp.wait()

    def fill_batch(bi, b):
        base = bi * (BATCH * LANES)
        bs = [bsp_v[pl.ds(base + rb * LANES, LANES)] for rb in range(BATCH)]

        @plsc.parallel_loop(0, CHUNKS, unroll=8)
        def chunk_body(c):
            d = d_v[pl.ds(c * LANES, LANES)]
            for rb in range(BATCH):
                buf_v[b, rb, pl.ds(c * LANES, LANES)] = plsc.load_gather(
                    col_v, [bs[rb] - d])

    def start_out(bi, b):
        pltpu.async_copy(buf_v.at[b],
                         out_hbm.at[pl.ds(r0 + bi * BATCH, BATCH)],
                         sems[b])

    def drain(b):
        pltpu.make_async_copy(buf_v.at[b],
                              out_hbm.at[pl.ds(r0, BATCH)],
                              sems[b]).wait()

    for b in range(NBUF):
        fill_batch(b, b)
        start_out(b, b)

    def outer(g, _):
        for b in range(NBUF):
            bi = g * NBUF + NBUF + b
            drain(b)
            fill_batch(bi, b)
            start_out(bi, b)
        return 0

    steady = (NBATCH - NBUF) // NBUF
    lax.fori_loop(0, steady, outer, 0)
    for t in range((NBATCH - NBUF) % NBUF):
        bi = NBUF + steady * NBUF + t
        b = bi % NBUF
        drain(b)
        fill_batch(bi, b)
        start_out(bi, b)

    for b in range(NBUF):
        drain(b)


def _sc_expand(tab):
    mesh = plsc.VectorSubcoreMesh(core_axis_name="c", subcore_axis_name="s",
                                  num_cores=NC, num_subcores=NS)
    fn = pl.kernel(
        _body,
        out_type=jax.ShapeDtypeStruct((ROWS, HW), jnp.float32),
        mesh=mesh,
        compiler_params=pltpu.CompilerParams(needs_layout_passes=False),
        scratch_types=[
            pltpu.VMEM((NCCH, CCH), jnp.int32),
            pltpu.VMEM((COLPAD,), jnp.float32),
            pltpu.VMEM((HW,), jnp.int32),
            pltpu.VMEM((RPW * LANES,), jnp.int32),
            pltpu.VMEM((NBUF, BATCH, HW), jnp.float32),
        ] + [pltpu.SemaphoreType.DMA] * (NBUF + 1),
    )
    return fn(tab, jnp.asarray(_COLIDX), jnp.asarray(_D), jnp.asarray(_BSP))


def _index_constants():
    k, off = 2 * W - 1, (H - 1) * (2 * W - 1) + (W - 1)
    j = np.arange(HW, dtype=np.int32)
    d = (j >> 5) * k + (j & 31)
    r = np.arange(ROWS, dtype=np.int32)
    i = r & (HW - 1)
    ball = (i >> 5) * k + (i & 31) + off
    bsp = np.broadcast_to(ball[:, None], (ROWS, LANES)).reshape(-1)
    t = np.arange(COLPAD, dtype=np.int32)
    colidx = np.where(t < SUB, t * NH, (SUB - 1) * NH)[None, :] \
        + np.arange(NH, dtype=np.int32)[:, None]
    return (d, np.ascontiguousarray(bsp),
            np.ascontiguousarray(colidx.reshape(NH, NCCH, CCH)))


_D, _BSP, _COLIDX = _index_constants()


def kernel(height, width, table):
    out = _sc_expand(table.reshape(-1))
    return out.reshape(NH, HW, HW)


# speedup vs baseline: 1.0308x; 1.0308x over previous
import jax
import jax.numpy as jnp
import numpy as np
from jax import lax
from jax.experimental import pallas as pl
from jax.experimental.pallas import tpu as pltpu
from jax.experimental.pallas import tpu_sc as plsc

H = W = 32
HW = H * W
NH = 16
SUB = (2 * H - 1) * (2 * W - 1)
NC, NS = 2, 16
NW = NC * NS
ROWS = NH * HW
RPW = ROWS // NW
LANES = 16
CHUNKS = HW // LANES
BATCH = 8
NBUF = 4
NBATCH = RPW // BATCH
NPIECE = 9
PR = SUB // NPIECE


def _body(tab_hbm, d16_hbm, bsp_hbm, out_hbm,
          sub_v, piece_v, d16_v, bsp_v, buf_v, *sems):
    cid = lax.axis_index("c")
    sid = lax.axis_index("s")
    wid = sid * NC + cid
    r0 = wid * RPW

    pltpu.sync_copy(d16_hbm, d16_v)
    pltpu.sync_copy(bsp_hbm.at[pl.ds(r0 * LANES, RPW * LANES)], bsp_v)

    base16 = lax.iota(jnp.int32, LANES) * SUB
    rv = jnp.zeros((LANES,), jnp.int32)
    for p in range(NPIECE):
        pltpu.sync_copy(tab_hbm.at[pl.ds(p * (PR * NH), PR * NH)], piece_v)

        def row_body(rl, rv):
            val = piece_v[pl.ds(rl * NH, NH)]
            plsc.store_scatter(sub_v, [base16 + rv], val)
            return rv + 1

        rv = lax.fori_loop(0, PR, row_body, rv, unroll=8)

    def fill_batch(bi, b):
        base = bi * (BATCH * LANES)
        bs = [bsp_v[pl.ds(base + rb * LANES, LANES)] for rb in range(BATCH)]

        @plsc.parallel_loop(0, CHUNKS, unroll=8)
        def chunk_body(c):
            d = d16_v[pl.ds(c * LANES, LANES)]
            for rb in range(BATCH):
                buf_v[b, rb, pl.ds(c * LANES, LANES)] = plsc.load_gather(
                    sub_v, [bs[rb] - d])

    def start_out(bi, b):
        pltpu.async_copy(buf_v.at[b],
                         out_hbm.at[pl.ds(r0 + bi * BATCH, BATCH)],
                         sems[b])

    def drain(b):
        pltpu.make_async_copy(buf_v.at[b],
                              out_hbm.at[pl.ds(r0, BATCH)],
                              sems[b]).wait()

    for b in range(NBUF):
        fill_batch(b, b)
        start_out(b, b)

    def outer(g, _):
        for b in range(NBUF):
            bi = g * NBUF + NBUF + b
            drain(b)
            fill_batch(bi, b)
            start_out(bi, b)
        return 0

    steady = (NBATCH - NBUF) // NBUF
    lax.fori_loop(0, steady, outer, 0)
    for t in range((NBATCH - NBUF) % NBUF):
        bi = NBUF + steady * NBUF + t
        b = bi % NBUF
        drain(b)
        fill_batch(bi, b)
        start_out(bi, b)

    for b in range(NBUF):
        drain(b)


def _sc_expand(tab):
    mesh = plsc.VectorSubcoreMesh(core_axis_name="c", subcore_axis_name="s",
                                  num_cores=NC, num_subcores=NS)
    fn = pl.kernel(
        _body,
        out_type=jax.ShapeDtypeStruct((ROWS, HW), jnp.float32),
        mesh=mesh,
        compiler_params=pltpu.CompilerParams(needs_layout_passes=False),
        scratch_types=[
            pltpu.VMEM((SUB * NH,), jnp.float32),
            pltpu.VMEM((PR * NH,), jnp.float32),
            pltpu.VMEM((HW,), jnp.int32),
            pltpu.VMEM((RPW * LANES,), jnp.int32),
            pltpu.VMEM((NBUF, BATCH, HW), jnp.float32),
        ] + [pltpu.SemaphoreType.DMA] * NBUF,
    )
    return fn(tab, jnp.asarray(_D16), jnp.asarray(_BSP))


def _index_constants():
    k, off = 2 * W - 1, (H - 1) * (2 * W - 1) + (W - 1)
    j = np.arange(HW, dtype=np.int32)
    d16 = (j >> 5) * k + (j & 31)
    r = np.arange(ROWS, dtype=np.int32)
    head = r >> 10
    i = r & (HW - 1)
    ball = (i >> 5) * k + (i & 31) + off + head * SUB
    bsp = np.broadcast_to(ball[:, None], (ROWS, LANES)).reshape(-1)
    return d16, np.ascontiguousarray(bsp)


_D16, _BSP = _index_constants()


def kernel(height, width, table):
    out = _sc_expand(table.reshape(-1))
    return out.reshape(NH, HW, HW)


# speedup vs baseline: 1.9738x; 1.9149x over previous
import jax
import jax.numpy as jnp
import numpy as np
from jax import lax
from jax.experimental import pallas as pl
from jax.experimental.pallas import tpu as pltpu
from jax.experimental.pallas import tpu_sc as plsc

H = W = 32
HW = H * W
NH = 16
SUB = (2 * H - 1) * (2 * W - 1)
NC, NS = 2, 16
NW = NC * NS
ROWS = NH * HW
RPW = ROWS // NW
LANES = 16
CHUNKS = HW // LANES
BATCH = 8
NBUF = 4
NBATCH = RPW // BATCH


def _body(sub_hbm, d16_hbm, bsp_hbm, out_hbm,
          sub_v, d16_v, bsp_v, buf_v, *sems):
    cid = lax.axis_index("c")
    sid = lax.axis_index("s")
    wid = sid * NC + cid
    r0 = wid * RPW

    pltpu.sync_copy(sub_hbm, sub_v)
    pltpu.sync_copy(d16_hbm, d16_v)
    pltpu.sync_copy(bsp_hbm.at[pl.ds(r0 * LANES, RPW * LANES)], bsp_v)

    def fill_batch(bi, b):
        base = bi * (BATCH * LANES)
        bs = [bsp_v[pl.ds(base + rb * LANES, LANES)] for rb in range(BATCH)]

        @plsc.parallel_loop(0, CHUNKS, unroll=8)
        def chunk_body(c):
            d = d16_v[pl.ds(c * LANES, LANES)]
            for rb in range(BATCH):
                buf_v[b, rb, pl.ds(c * LANES, LANES)] = plsc.load_gather(
                    sub_v, [bs[rb] - d])

    def start_out(bi, b):
        pltpu.async_copy(buf_v.at[b],
                         out_hbm.at[pl.ds(r0 + bi * BATCH, BATCH)],
                         sems[b])

    def drain(b):
        pltpu.make_async_copy(buf_v.at[b],
                              out_hbm.at[pl.ds(r0, BATCH)],
                              sems[b]).wait()

    for b in range(NBUF):
        fill_batch(b, b)
        start_out(b, b)

    def outer(g, _):
        for b in range(NBUF):
            bi = g * NBUF + NBUF + b
            drain(b)
            fill_batch(bi, b)
            start_out(bi, b)
        return 0

    steady = (NBATCH - NBUF) // NBUF
    lax.fori_loop(0, steady, outer, 0)
    for t in range((NBATCH - NBUF) % NBUF):
        bi = NBUF + steady * NBUF + t
        b = bi % NBUF
        drain(b)
        fill_batch(bi, b)
        start_out(bi, b)

    for b in range(NBUF):
        drain(b)


def _sc_expand(sub, d16, bsp):
    mesh = plsc.VectorSubcoreMesh(core_axis_name="c", subcore_axis_name="s",
                                  num_cores=NC, num_subcores=NS)
    fn = pl.kernel(
        _body,
        out_type=jax.ShapeDtypeStruct((ROWS, HW), jnp.float32),
        mesh=mesh,
        compiler_params=pltpu.CompilerParams(needs_layout_passes=False),
        scratch_types=[
            pltpu.VMEM((SUB * (NH + 1),), jnp.float32),
            pltpu.VMEM((HW,), jnp.int32),
            pltpu.VMEM((RPW * LANES,), jnp.int32),
            pltpu.VMEM((NBUF, BATCH, HW), jnp.float32),
        ] + [pltpu.SemaphoreType.DMA] * NBUF,
    )
    return fn(sub, d16, bsp)


def _index_constants():
    k, off = 2 * W - 1, (H - 1) * (2 * W - 1) + (W - 1)
    j = np.arange(HW, dtype=np.int32)
    d16 = ((j >> 5) * k + (j & 31)) * (NH + 1)
    r = np.arange(ROWS, dtype=np.int32)
    head = r >> 10
    i = r & (HW - 1)
    ball = ((i >> 5) * k + (i & 31) + off) * (NH + 1) + head
    bsp = np.broadcast_to(ball[:, None], (ROWS, LANES)).reshape(-1)
    return d16, np.ascontiguousarray(bsp)


_D16, _BSP = _index_constants()


def kernel(height, width, table):
    sub = jnp.pad(table[:SUB], ((0, 0), (0, 1))).reshape(-1)
    out = _sc_expand(sub, jnp.asarray(_D16), jnp.asarray(_BSP))
    return out.reshape(NH, HW, HW)


# speedup vs baseline: 1.9966x; 1.0116x over previous
import jax
import jax.numpy as jnp
import numpy as np
from jax import lax
from jax.experimental import pallas as pl
from jax.experimental.pallas import tpu as pltpu
from jax.experimental.pallas import tpu_sc as plsc

H = W = 32
HW = H * W
NH = 16
SUB = (2 * H - 1) * (2 * W - 1)
NC, NS = 2, 16
NW = NC * NS
ROWS = NH * HW
RPW = ROWS // NW
LANES = 16
CHUNKS = HW // LANES
BATCH = 8
NBUF = 4
NBATCH = RPW // BATCH


def _body(sub_hbm, d16_hbm, bsp_hbm, out_hbm,
          sub_v, d16_v, bsp_v, buf_v, *sems):
    cid = lax.axis_index("c")
    sid = lax.axis_index("s")
    wid = sid * NC + cid
    r0 = wid * RPW

    pltpu.sync_copy(sub_hbm, sub_v)
    pltpu.sync_copy(d16_hbm, d16_v)
    pltpu.sync_copy(bsp_hbm.at[pl.ds(r0 * LANES, RPW * LANES)], bsp_v)

    def fill_batch(bi, b):
        base = bi * (BATCH * LANES)
        bs = [bsp_v[pl.ds(base + rb * LANES, LANES)] for rb in range(BATCH)]

        @plsc.parallel_loop(0, CHUNKS, unroll=8)
        def chunk_body(c):
            d = d16_v[pl.ds(c * LANES, LANES)]
            for rb in range(BATCH):
                buf_v[b, rb, pl.ds(c * LANES, LANES)] = plsc.load_gather(
                    sub_v, [bs[rb] - d])

    def start_out(bi, b):
        pltpu.async_copy(buf_v.at[b],
                         out_hbm.at[pl.ds(r0 + bi * BATCH, BATCH)],
                         sems[b])

    def drain(b):
        pltpu.make_async_copy(buf_v.at[b],
                              out_hbm.at[pl.ds(r0, BATCH)],
                              sems[b]).wait()

    for b in range(NBUF):
        fill_batch(b, b)
        start_out(b, b)

    def outer(g, _):
        for b in range(NBUF):
            bi = g * NBUF + NBUF + b
            drain(b)
            fill_batch(bi, b)
            start_out(bi, b)
        return 0

    steady = (NBATCH - NBUF) // NBUF
    lax.fori_loop(0, steady, outer, 0)
    for t in range((NBATCH - NBUF) % NBUF):
        bi = NBUF + steady * NBUF + t
        b = bi % NBUF
        drain(b)
        fill_batch(bi, b)
        start_out(bi, b)

    for b in range(NBUF):
        drain(b)


def _sc_expand(sub, d16, bsp):
    mesh = plsc.VectorSubcoreMesh(core_axis_name="c", subcore_axis_name="s",
                                  num_cores=NC, num_subcores=NS)
    fn = pl.kernel(
        _body,
        out_type=jax.ShapeDtypeStruct((ROWS, HW), jnp.float32),
        mesh=mesh,
        compiler_params=pltpu.CompilerParams(needs_layout_passes=False,
                                             disable_bounds_checks=True),
        scratch_types=[
            pltpu.VMEM((SUB * NH,), jnp.float32),
            pltpu.VMEM((HW,), jnp.int32),
            pltpu.VMEM((RPW * LANES,), jnp.int32),
            pltpu.VMEM((NBUF, BATCH, HW), jnp.float32),
        ] + [pltpu.SemaphoreType.DMA] * NBUF,
    )
    return fn(sub, d16, bsp)


def _index_constants():
    k, off = 2 * W - 1, (H - 1) * (2 * W - 1) + (W - 1)
    j = np.arange(HW, dtype=np.int32)
    d16 = (j >> 5) * k + (j & 31)
    r = np.arange(ROWS, dtype=np.int32)
    head = r >> 10
    i = r & (HW - 1)
    ball = (i >> 5) * k + (i & 31) + off + head * SUB
    bsp = np.broadcast_to(ball[:, None], (ROWS, LANES)).reshape(-1)
    return jnp.asarray(d16), jnp.asarray(bsp.copy())


_D16, _BSP = _index_constants()


def kernel(height, width, table):
    sub = jnp.transpose(table[:SUB]).reshape(-1)
    out = _sc_expand(sub, _D16, _BSP)
    return out.reshape(NH, HW, HW)
